# Initial kernel scaffold; baseline (speedup 1.0000x reference)
import jax
import jax.numpy as jnp
from jax.experimental import pallas as pl


def kernel(x, edge_index, W1l, b1l, W1r, W2l, b2l, W2r):
    raise NotImplementedError("write your pallas kernel here")


# speedup vs baseline: 3.8498x; 3.8498x over previous
import functools
import jax
import jax.numpy as jnp
from jax import lax
from jax.experimental import pallas as pl
from jax.experimental.pallas import tpu as pltpu
from jax.experimental.pallas import tpu_sc as plsc

N_NODES = 10000
N_EDGES = 160000
DIM = 256
HIDDEN = 512
QUART = DIM // 4
PADC = 16

N_SUB = 16
NP = 10240
CHUNK = 80
EPT1 = N_EDGES // N_SUB
NCH1 = EPT1 // CHUNK
RPT = NP // N_SUB
RCH = RPT // 5

_MESH = plsc.VectorSubcoreMesh(core_axis_name="c", subcore_axis_name="s")


def _zero_fill_2d(ref, nrows, ncols):
    z = jnp.zeros((16,), jnp.float32)

    def row(r, carry):
        for j in range(ncols // 16):
            ref[r, pl.ds(j * 16, 16)] = z
        return carry

    lax.fori_loop(0, nrows, row, 0)


@functools.partial(
    pl.kernel,
    out_type=[
        jax.ShapeDtypeStruct((4 * NP, QUART), jnp.float32),
        jax.ShapeDtypeStruct((NP, PADC), jnp.float32),
    ],
    mesh=_MESH,
    compiler_params=pltpu.CompilerParams(use_tc_tiling_on_sc=False),
    scratch_types=[
        pltpu.VMEM((CHUNK,), jnp.int32),
        pltpu.VMEM((CHUNK,), jnp.int32),
        pltpu.VMEM((CHUNK, QUART), jnp.float32),
        pltpu.VMEM((CHUNK, PADC), jnp.float32),
        pltpu.VMEM((RCH, QUART), jnp.float32),
        pltpu.VMEM((RPT, PADC), jnp.float32),
        pltpu.VMEM_SHARED((NP, QUART), jnp.float32),
        pltpu.VMEM_SHARED((NP, PADC), jnp.float32),
        pltpu.SemaphoreType.DMA,
    ],
)
def _agg1(xs_hbm, src_hbm, dst_hbm, out_hbm, cnt_hbm,
          src_v, dst_v, rows_v, ones_v, tmp_v, tmp16_v, acc_s, cnt_s, sem):
    c = lax.axis_index("c")
    s = lax.axis_index("s")

    _zero_fill_2d(tmp_v, RCH, QUART)
    _zero_fill_2d(tmp16_v, RPT, PADC)
    one = jnp.ones((16,), jnp.float32)

    def ones_row(r, carry):
        ones_v[r, pl.ds(0, PADC)] = one
        return carry

    lax.fori_loop(0, CHUNK, ones_row, 0)

    base0 = s * EPT1

    for p in range(2):
        q = 2 * p + c
        off = q * NP

        for k in range(5):
            pltpu.sync_copy(tmp_v, acc_s.at[pl.ds(s * RPT + k * RCH, RCH)])
        if p == 0:
            pltpu.sync_copy(tmp16_v, cnt_s.at[pl.ds(s * RPT, RPT)])
        plsc.subcore_barrier()

        def chunk(i, carry):
            base = base0 + i * CHUNK
            pltpu.sync_copy(src_hbm.at[pl.ds(base, CHUNK)], src_v)
            pltpu.sync_copy(dst_hbm.at[pl.ds(base, CHUNK)], dst_v)
            for j in range(CHUNK // 16):
                src_v[pl.ds(j * 16, 16)] = src_v[pl.ds(j * 16, 16)] + off
            pltpu.async_copy(xs_hbm.at[src_v], rows_v, sem).wait()
            pltpu.sync_copy(rows_v, acc_s.at[dst_v], add=True)
            if p == 0:
                pltpu.sync_copy(ones_v, cnt_s.at[dst_v], add=True)
            return carry

        lax.fori_loop(0, NCH1, chunk, 0)
        plsc.subcore_barrier()

        for k in range(5):
            r0 = s * RPT + k * RCH
            pltpu.sync_copy(acc_s.at[pl.ds(r0, RCH)], tmp_v)
            pltpu.sync_copy(tmp_v, out_hbm.at[pl.ds(off + r0, RCH)])
            if p == 0:
                _zero_fill_2d(tmp_v, RCH, QUART)

        @pl.when((c == 0) & (p == 0))
        def _():
            pltpu.sync_copy(cnt_s.at[pl.ds(s * RPT, RPT)], tmp16_v)
            pltpu.sync_copy(tmp16_v, cnt_hbm.at[pl.ds(s * RPT, RPT)])


_BM = 640


def _dense_body(x_ref, sm_ref, cnt_ref, w1l_ref, w1r_ref, b1_ref,
                w2l_ref, w2r_ref, b2_ref, p_ref, q_ref):
    cnt = jnp.maximum(cnt_ref[:, :1], 1.0)
    mean = sm_ref[...] / cnt
    h = (jnp.dot(mean, w1l_ref[...], preferred_element_type=jnp.float32)
         + jnp.dot(x_ref[...], w1r_ref[...], preferred_element_type=jnp.float32)
         + b1_ref[...])
    h = jnp.maximum(h, 0.0)
    p_ref[...] = jnp.dot(h, w2l_ref[...], preferred_element_type=jnp.float32)
    q_ref[...] = (jnp.dot(h, w2r_ref[...], preferred_element_type=jnp.float32)
                  + b2_ref[...])


_dense = pl.pallas_call(
    _dense_body,
    grid=(NP // _BM,),
    in_specs=[
        pl.BlockSpec((_BM, DIM), lambda i: (i, 0)),
        pl.BlockSpec((_BM, DIM), lambda i: (i, 0)),
        pl.BlockSpec((_BM, PADC), lambda i: (i, 0)),
        pl.BlockSpec((DIM, HIDDEN), lambda i: (0, 0)),
        pl.BlockSpec((DIM, HIDDEN), lambda i: (0, 0)),
        pl.BlockSpec((1, HIDDEN), lambda i: (0, 0)),
        pl.BlockSpec((HIDDEN, PADC), lambda i: (0, 0)),
        pl.BlockSpec((HIDDEN, PADC), lambda i: (0, 0)),
        pl.BlockSpec((1, PADC), lambda i: (0, 0)),
    ],
    out_specs=[
        pl.BlockSpec((_BM, PADC), lambda i: (i, 0)),
        pl.BlockSpec((_BM, PADC), lambda i: (i, 0)),
    ],
    out_shape=[
        jax.ShapeDtypeStruct((NP, PADC), jnp.float32),
        jax.ShapeDtypeStruct((NP, PADC), jnp.float32),
    ],
)


EPT2 = N_EDGES // N_SUB
NCH2 = EPT2 // CHUNK


@functools.partial(
    pl.kernel,
    out_type=jax.ShapeDtypeStruct((NP, PADC), jnp.float32),
    mesh=_MESH,
    compiler_params=pltpu.CompilerParams(use_tc_tiling_on_sc=False),
    scratch_types=[
        pltpu.VMEM((CHUNK,), jnp.int32),
        pltpu.VMEM((CHUNK,), jnp.int32),
        pltpu.VMEM((CHUNK, PADC), jnp.float32),
        pltpu.VMEM((RPT, PADC), jnp.float32),
        pltpu.VMEM((RPT, PADC), jnp.float32),
        pltpu.VMEM((RPT, PADC), jnp.float32),
        pltpu.VMEM_SHARED((NP, PADC), jnp.float32),
        pltpu.SemaphoreType.DMA,
    ],
)
def _agg2(p_hbm, q_hbm, cnt_hbm, src_hbm, dst_hbm, out_hbm,
          src_v, dst_v, rows_v, a_v, c_v, q_v, acc_s, sem):
    c = lax.axis_index("c")
    s = lax.axis_index("s")

    @pl.when(c == 0)
    def _():
        _zero_fill_2d(a_v, RPT, PADC)
        pltpu.sync_copy(a_v, acc_s.at[pl.ds(s * RPT, RPT)])
        plsc.subcore_barrier()

        base0 = s * EPT2

        def chunk(i, carry):
            base = base0 + i * CHUNK
            pltpu.sync_copy(src_hbm.at[pl.ds(base, CHUNK)], src_v)
            pltpu.sync_copy(dst_hbm.at[pl.ds(base, CHUNK)], dst_v)
            pltpu.async_copy(p_hbm.at[src_v], rows_v, sem).wait()
            pltpu.sync_copy(rows_v, acc_s.at[dst_v], add=True)
            return carry

        lax.fori_loop(0, NCH2, chunk, 0)
        plsc.subcore_barrier()

        r0 = s * RPT
        pltpu.sync_copy(acc_s.at[pl.ds(r0, RPT)], a_v)
        pltpu.sync_copy(cnt_hbm.at[pl.ds(r0, RPT)], c_v)
        pltpu.sync_copy(q_hbm.at[pl.ds(r0, RPT)], q_v)

        def row(r, carry):
            agg = a_v[r, pl.ds(0, PADC)]
            cc = jnp.maximum(c_v[r, pl.ds(0, PADC)], 1.0)
            q_v[r, pl.ds(0, PADC)] = agg / cc + q_v[r, pl.ds(0, PADC)]
            return carry

        lax.fori_loop(0, RPT, row, 0)
        pltpu.sync_copy(q_v, out_hbm.at[pl.ds(r0, RPT)])


def kernel(x, edge_index, W1l, b1l, W1r, W2l, b2l, W2r):
    src = edge_index[0].astype(jnp.int32)
    dst = edge_index[1].astype(jnp.int32)

    xp = jnp.pad(x, ((0, NP - N_NODES), (0, 0)))
    xs = jnp.concatenate([xp[:, q * QUART:(q + 1) * QUART] for q in range(4)],
                         axis=0)
    summed4, cnt = _agg1(xs, src, dst)
    summed = jnp.concatenate([summed4[q * NP:(q + 1) * NP] for q in range(4)],
                             axis=1)

    nc = W2l.shape[0]
    padw = jnp.zeros((PADC - nc, HIDDEN), jnp.float32)
    w2l_t = jnp.concatenate([W2l, padw], axis=0).T
    w2r_t = jnp.concatenate([W2r, padw], axis=0).T
    b2p = jnp.concatenate([b2l, jnp.zeros((PADC - nc,), jnp.float32)])[None]

    p16, q16 = _dense(xp, summed, cnt, W1l.T, W1r.T, b1l[None],
                      w2l_t, w2r_t, b2p)
    out16 = _agg2(p16, q16, cnt, src, dst)
    return out16[:N_NODES, :nc]


# speedup vs baseline: 8.1587x; 2.1193x over previous
import functools
import jax
import jax.numpy as jnp
from jax import lax
from jax.experimental import pallas as pl
from jax.experimental.pallas import tpu as pltpu
from jax.experimental.pallas import tpu_sc as plsc

N_NODES = 10000
N_EDGES = 160000
DIM = 256
HIDDEN = 512
QUART = DIM // 4
PADC = 16

N_SUB = 16
NP = 10240
CHUNK = 80
EPT = N_EDGES // N_SUB
NCH = EPT // CHUNK
NPAIR = (NCH - 1) // 2
RPT = NP // N_SUB
RCH = RPT // 5

_MESH = plsc.VectorSubcoreMesh(core_axis_name="c", subcore_axis_name="s")
_SC_PARAMS = pltpu.CompilerParams(use_tc_tiling_on_sc=False)


def _zero_fill_2d(ref, nrows, ncols):
    z = jnp.zeros((16,), jnp.float32)

    def row(r, carry):
        for j in range(ncols // 16):
            ref[r, pl.ds(j * 16, 16)] = z
        return carry

    lax.fori_loop(0, nrows, row, 0)


@functools.partial(
    pl.kernel,
    out_type=[
        jax.ShapeDtypeStruct((4 * NP, QUART), jnp.float32),
        jax.ShapeDtypeStruct((NP, PADC), jnp.float32),
    ],
    mesh=_MESH,
    compiler_params=_SC_PARAMS,
    scratch_types=[
        pltpu.VMEM((NCH, CHUNK), jnp.int32),
        pltpu.VMEM((NCH, CHUNK), jnp.int32),
        pltpu.VMEM((CHUNK,), jnp.int32),
        pltpu.VMEM((CHUNK,), jnp.int32),
        pltpu.VMEM((CHUNK, QUART), jnp.float32),
        pltpu.VMEM((CHUNK, QUART), jnp.float32),
        pltpu.VMEM((CHUNK, PADC), jnp.float32),
        pltpu.VMEM((RCH, QUART), jnp.float32),
        pltpu.VMEM((RPT, PADC), jnp.float32),
        pltpu.VMEM_SHARED((NP, QUART), jnp.float32),
        pltpu.VMEM_SHARED((NP, PADC), jnp.float32),
        pltpu.SemaphoreType.DMA,
        pltpu.SemaphoreType.DMA,
    ],
)
def _agg1(xs_hbm, src2_hbm, dst2_hbm, out_hbm, cnt_hbm,
          srcb_v, dstb_v, idx0_v, idx1_v, rows0_v, rows1_v, ones_v,
          tmp_v, tmp16_v, acc_s, cnt_s, gsem0, gsem1):
    c = lax.axis_index("c")
    s = lax.axis_index("s")

    _zero_fill_2d(tmp_v, RCH, QUART)
    _zero_fill_2d(tmp16_v, RPT, PADC)
    one = jnp.ones((16,), jnp.float32)

    def ones_row(r, carry):
        ones_v[r, pl.ds(0, PADC)] = one
        return carry

    lax.fori_loop(0, CHUNK, ones_row, 0)

    pltpu.sync_copy(src2_hbm.at[pl.ds(s * NCH, NCH)], srcb_v)
    pltpu.sync_copy(dst2_hbm.at[pl.ds(s * NCH, NCH)], dstb_v)

    for p in range(2):
        qq = 2 * p + c

        def fill_idx(ci, dest):
            for j in range(CHUNK // 16):
                dest[pl.ds(j * 16, 16)] = srcb_v[ci, pl.ds(j * 16, 16)] * 4 + qq

        for k in range(5):
            pltpu.sync_copy(tmp_v, acc_s.at[pl.ds(s * RPT + k * RCH, RCH)])
        if p == 0:
            pltpu.sync_copy(tmp16_v, cnt_s.at[pl.ds(s * RPT, RPT)])
        plsc.subcore_barrier()

        fill_idx(0, idx0_v)
        pltpu.async_copy(xs_hbm.at[idx0_v], rows0_v, gsem0)

        def pair(g, carry):
            c0 = 2 * g
            fill_idx(c0 + 1, idx1_v)
            pltpu.make_async_copy(xs_hbm.at[idx0_v], rows0_v, gsem0).wait()
            pltpu.async_copy(xs_hbm.at[idx1_v], rows1_v, gsem1)
            pltpu.sync_copy(rows0_v, acc_s.at[dstb_v.at[c0]], add=True)
            if p == 0:
                pltpu.sync_copy(ones_v, cnt_s.at[dstb_v.at[c0]], add=True)
            fill_idx(c0 + 2, idx0_v)
            pltpu.make_async_copy(xs_hbm.at[idx1_v], rows1_v, gsem1).wait()
            pltpu.async_copy(xs_hbm.at[idx0_v], rows0_v, gsem0)
            pltpu.sync_copy(rows1_v, acc_s.at[dstb_v.at[c0 + 1]], add=True)
            if p == 0:
                pltpu.sync_copy(ones_v, cnt_s.at[dstb_v.at[c0 + 1]], add=True)
            return carry

        lax.fori_loop(0, NPAIR, pair, 0)
        pltpu.make_async_copy(xs_hbm.at[idx0_v], rows0_v, gsem0).wait()
        pltpu.sync_copy(rows0_v, acc_s.at[dstb_v.at[NCH - 1]], add=True)
        if p == 0:
            pltpu.sync_copy(ones_v, cnt_s.at[dstb_v.at[NCH - 1]], add=True)
        plsc.subcore_barrier()

        for k in range(5):
            r0 = s * RPT + k * RCH
            pltpu.sync_copy(acc_s.at[pl.ds(r0, RCH)], tmp_v)
            pltpu.sync_copy(tmp_v, out_hbm.at[pl.ds(qq * NP + r0, RCH)])
        if p == 0:
            _zero_fill_2d(tmp_v, RCH, QUART)

            @pl.when(c == 0)
            def _():
                pltpu.sync_copy(cnt_s.at[pl.ds(s * RPT, RPT)], tmp16_v)
                pltpu.sync_copy(tmp16_v, cnt_hbm.at[pl.ds(s * RPT, RPT)])


_BM = 640


def _dense_body(x_ref, s0_ref, s1_ref, s2_ref, s3_ref, cnt_ref,
                w1l_ref, w1r_ref, b1_ref, w2l_ref, w2r_ref, b2_ref,
                p_ref, q_ref):
    inv = 1.0 / jnp.maximum(cnt_ref[:, :1], 1.0)
    mean = jnp.concatenate(
        [s0_ref[...], s1_ref[...], s2_ref[...], s3_ref[...]], axis=1) * inv
    h = (jnp.dot(mean, w1l_ref[...], preferred_element_type=jnp.float32)
         + jnp.dot(x_ref[...], w1r_ref[...], preferred_element_type=jnp.float32)
         + b1_ref[...])
    h = jnp.maximum(h, 0.0)
    p_ref[...] = jnp.dot(h, w2l_ref[...], preferred_element_type=jnp.float32)
    q_ref[...] = (jnp.dot(h, w2r_ref[...], preferred_element_type=jnp.float32)
                  + b2_ref[...])


def _make_sum_spec(q):
    return pl.BlockSpec((_BM, QUART), lambda i, q=q: (q * (NP // _BM) + i, 0))


_dense = pl.pallas_call(
    _dense_body,
    grid=(NP // _BM,),
    in_specs=[
        pl.BlockSpec((_BM, DIM), lambda i: (i, 0)),
        _make_sum_spec(0),
        _make_sum_spec(1),
        _make_sum_spec(2),
        _make_sum_spec(3),
        pl.BlockSpec((_BM, PADC), lambda i: (i, 0)),
        pl.BlockSpec((DIM, HIDDEN), lambda i: (0, 0)),
        pl.BlockSpec((DIM, HIDDEN), lambda i: (0, 0)),
        pl.BlockSpec((1, HIDDEN), lambda i: (0, 0)),
        pl.BlockSpec((HIDDEN, PADC), lambda i: (0, 0)),
        pl.BlockSpec((HIDDEN, PADC), lambda i: (0, 0)),
        pl.BlockSpec((1, PADC), lambda i: (0, 0)),
    ],
    out_specs=[
        pl.BlockSpec((_BM, PADC), lambda i: (i, 0)),
        pl.BlockSpec((_BM, PADC), lambda i: (i, 0)),
    ],
    out_shape=[
        jax.ShapeDtypeStruct((NP, PADC), jnp.float32),
        jax.ShapeDtypeStruct((NP, PADC), jnp.float32),
    ],
)


@functools.partial(
    pl.kernel,
    out_type=jax.ShapeDtypeStruct((NP, PADC), jnp.float32),
    mesh=_MESH,
    compiler_params=_SC_PARAMS,
    scratch_types=[
        pltpu.VMEM((NCH, CHUNK), jnp.int32),
        pltpu.VMEM((NCH, CHUNK), jnp.int32),
        pltpu.VMEM((CHUNK, PADC), jnp.float32),
        pltpu.VMEM((CHUNK, PADC), jnp.float32),
        pltpu.VMEM((RPT, PADC), jnp.float32),
        pltpu.VMEM((RPT, PADC), jnp.float32),
        pltpu.VMEM((RPT, PADC), jnp.float32),
        pltpu.VMEM_SHARED((NP, PADC), jnp.float32),
        pltpu.SemaphoreType.DMA,
        pltpu.SemaphoreType.DMA,
    ],
)
def _agg2(p_hbm, q_hbm, cnt_hbm, src2_hbm, dst2_hbm, out_hbm,
          srcb_v, dstb_v, rows0_v, rows1_v, a_v, c_v, q_v, acc_s,
          gsem0, gsem1):
    c = lax.axis_index("c")
    s = lax.axis_index("s")

    @pl.when(c == 0)
    def _():
        _zero_fill_2d(a_v, RPT, PADC)
        pltpu.sync_copy(a_v, acc_s.at[pl.ds(s * RPT, RPT)])
        pltpu.sync_copy(src2_hbm.at[pl.ds(s * NCH, NCH)], srcb_v)
        pltpu.sync_copy(dst2_hbm.at[pl.ds(s * NCH, NCH)], dstb_v)
        plsc.subcore_barrier()

        pltpu.async_copy(p_hbm.at[srcb_v.at[0]], rows0_v, gsem0)

        def pair(g, carry):
            c0 = 2 * g
            pltpu.make_async_copy(p_hbm.at[srcb_v.at[c0]], rows0_v,
                                  gsem0).wait()
            pltpu.async_copy(p_hbm.at[srcb_v.at[c0 + 1]], rows1_v, gsem1)
            pltpu.sync_copy(rows0_v, acc_s.at[dstb_v.at[c0]], add=True)
            pltpu.make_async_copy(p_hbm.at[srcb_v.at[c0 + 1]], rows1_v,
                                  gsem1).wait()
            pltpu.async_copy(p_hbm.at[srcb_v.at[c0 + 2]], rows0_v, gsem0)
            pltpu.sync_copy(rows1_v, acc_s.at[dstb_v.at[c0 + 1]], add=True)
            return carry

        lax.fori_loop(0, NPAIR, pair, 0)
        pltpu.make_async_copy(p_hbm.at[srcb_v.at[NCH - 1]], rows0_v,
                              gsem0).wait()
        pltpu.sync_copy(rows0_v, acc_s.at[dstb_v.at[NCH - 1]], add=True)
        plsc.subcore_barrier()

        r0 = s * RPT
        pltpu.sync_copy(acc_s.at[pl.ds(r0, RPT)], a_v)
        pltpu.sync_copy(cnt_hbm.at[pl.ds(r0, RPT)], c_v)
        pltpu.sync_copy(q_hbm.at[pl.ds(r0, RPT)], q_v)

        def row(r, carry):
            agg = a_v[r, pl.ds(0, PADC)]
            cc = jnp.maximum(c_v[r, pl.ds(0, PADC)], 1.0)
            q_v[r, pl.ds(0, PADC)] = agg / cc + q_v[r, pl.ds(0, PADC)]
            return carry

        lax.fori_loop(0, RPT, row, 0)
        pltpu.sync_copy(q_v, out_hbm.at[pl.ds(r0, RPT)])


def kernel(x, edge_index, W1l, b1l, W1r, W2l, b2l, W2r):
    src = edge_index[0].astype(jnp.int32)
    dst = edge_index[1].astype(jnp.int32)

    xs = x.reshape(4 * N_NODES, QUART)
    src2 = src.reshape(N_EDGES // CHUNK, CHUNK)
    dst2 = dst.reshape(N_EDGES // CHUNK, CHUNK)

    summed4, cnt = _agg1(xs, src2, dst2)

    nc = W2l.shape[0]
    padw = jnp.zeros((PADC - nc, HIDDEN), jnp.float32)
    w2l_t = jnp.concatenate([W2l, padw], axis=0).T
    w2r_t = jnp.concatenate([W2r, padw], axis=0).T
    b2p = jnp.concatenate([b2l, jnp.zeros((PADC - nc,), jnp.float32)])[None]

    p16, q16 = _dense(x, summed4, summed4, summed4, summed4, cnt,
                      W1l.T, W1r.T, b1l[None], w2l_t, w2r_t, b2p)
    out16 = _agg2(p16, q16, cnt, src2, dst2)
    return out16[:N_NODES, :nc]
